# Initial kernel scaffold; baseline (speedup 1.0000x reference)
import functools

import jax
import jax.numpy as jnp
from jax import lax
from jax.experimental import pallas as pl
from jax.experimental.pallas import tpu as pltpu
from jax.experimental.pallas import tpu_sc as plsc

N = 10000
E = 320000
D = 128
H = 128
O = 64
G = 64
EPS = 1e-5

NC = 2
NS = 16
CHUNK = 128
NCHUNKS = -(-E // (NC * NS * CHUNK))
EPT = NCHUNKS * CHUNK
EPAD = EPT * NC * NS
NROW = N // NS
NACC = N + 16

_mesh = plsc.VectorSubcoreMesh(
    core_axis_name="c", subcore_axis_name="s", num_cores=NC, num_subcores=NS)


def _make_spmm(width):
  @functools.partial(
      pl.kernel,
      out_type=jax.ShapeDtypeStruct((NC, N, width), jnp.float32),
      mesh=_mesh,
      scratch_types=[
          pltpu.VMEM((CHUNK,), jnp.int32),
          pltpu.VMEM((CHUNK,), jnp.int32),
          pltpu.VMEM((CHUNK, width), jnp.float32),
          pltpu.VMEM_SHARED((NACC, width), jnp.float32),
          pltpu.SemaphoreType.DMA,
      ],
  )
  def spmm(src_hbm, dst_hbm, p_hbm, zeros_hbm, out_hbm,
           src_v, dst_v, rows_v, acc, sem):
    cid = lax.axis_index("c")
    sid = lax.axis_index("s")
    pltpu.sync_copy(zeros_hbm.at[pl.ds(sid * NROW, NROW)],
                    acc.at[pl.ds(sid * NROW, NROW)])
    plsc.subcore_barrier()
    base = (cid * NS + sid) * EPT

    def body(i, carry):
      off = base + i * CHUNK
      pltpu.sync_copy(src_hbm.at[pl.ds(off, CHUNK)], src_v)
      pltpu.sync_copy(dst_hbm.at[pl.ds(off, CHUNK)], dst_v)
      pltpu.async_copy(p_hbm.at[src_v], rows_v, sem).wait()
      pltpu.sync_copy(rows_v, acc.at[dst_v], add=True)
      return carry

    lax.fori_loop(0, NCHUNKS, body, 0)
    plsc.subcore_barrier()
    pltpu.sync_copy(acc.at[pl.ds(sid * NROW, NROW)],
                    out_hbm.at[cid, pl.ds(sid * NROW, NROW)])

  return spmm


_spmm_h = _make_spmm(H)
_spmm_o = _make_spmm(O)


@functools.partial(
    pl.kernel,
    out_type=jax.ShapeDtypeStruct((NC, N, 16), jnp.float32),
    mesh=_mesh,
    scratch_types=[
        pltpu.VMEM((CHUNK,), jnp.int32),
        pltpu.VMEM((CHUNK, 16), jnp.float32),
        pltpu.VMEM_SHARED((NACC, 16), jnp.float32),
    ],
)
def _deg_kernel(dst_hbm, ones_hbm, zeros_hbm, out_hbm, dst_v, ones_v, acc):
  cid = lax.axis_index("c")
  sid = lax.axis_index("s")
  pltpu.sync_copy(ones_hbm, ones_v)
  pltpu.sync_copy(zeros_hbm.at[pl.ds(sid * NROW, NROW)],
                  acc.at[pl.ds(sid * NROW, NROW)])
  plsc.subcore_barrier()
  base = (cid * NS + sid) * EPT

  def body(i, carry):
    off = base + i * CHUNK
    pltpu.sync_copy(dst_hbm.at[pl.ds(off, CHUNK)], dst_v)
    pltpu.sync_copy(ones_v, acc.at[dst_v], add=True)
    return carry

  lax.fori_loop(0, NCHUNKS, body, 0)
  plsc.subcore_barrier()
  pltpu.sync_copy(acc.at[pl.ds(sid * NROW, NROW)],
                  out_hbm.at[cid, pl.ds(sid * NROW, NROW)])


RB = 400
NB = N // RB
PB = 200
NPB = N // PB


def _tc1_body(dp_ref, x_ref, w_ref, b_ref, dinv_ref, p_ref):
  dp = dp_ref[...]
  deg = 1.0 + dp[0, :, 0:1] + dp[1, :, 0:1]
  dinv = lax.rsqrt(deg)
  dinv_ref[...] = dinv
  h = jnp.dot(x_ref[...], w_ref[...], preferred_element_type=jnp.float32)
  p_ref[...] = dinv * (h + b_ref[...])


def _tc1(degp, x, w1, b1):
  return pl.pallas_call(
      _tc1_body,
      grid=(NB,),
      in_specs=[
          pl.BlockSpec((NC, RB, 16), lambda i: (0, i, 0)),
          pl.BlockSpec((RB, D), lambda i: (i, 0)),
          pl.BlockSpec((D, H), lambda i: (0, 0)),
          pl.BlockSpec((1, H), lambda i: (0, 0)),
      ],
      out_specs=[
          pl.BlockSpec((RB, 1), lambda i: (i, 0)),
          pl.BlockSpec((RB, H), lambda i: (i, 0)),
      ],
      out_shape=[
          jax.ShapeDtypeStruct((N, 1), jnp.float32),
          jax.ShapeDtypeStruct((N, H), jnp.float32),
      ],
  )(degp, x, w1, b1)


def _mid_body(s_ref, pprev_ref, dinv_ref, g_ref, be_ref, w_ref, b_ref,
              pnext_ref):
  s = s_ref[...]
  dinv = dinv_ref[...]
  tot = (s[0] + s[1] + pprev_ref[...]) * dinv
  z = tot / jnp.sqrt(1.0 + EPS) * g_ref[...] + be_ref[...]
  z = jnp.where(z >= 0, z, 0.01 * z)
  h = jnp.dot(z, w_ref[...], preferred_element_type=jnp.float32)
  pnext_ref[...] = dinv * (h + b_ref[...])


def _mid(s, pprev, dinv, g, be, w, b, width_out):
  return pl.pallas_call(
      _mid_body,
      grid=(NB,),
      in_specs=[
          pl.BlockSpec((NC, RB, H), lambda i: (0, i, 0)),
          pl.BlockSpec((RB, H), lambda i: (i, 0)),
          pl.BlockSpec((RB, 1), lambda i: (i, 0)),
          pl.BlockSpec((1, H), lambda i: (0, 0)),
          pl.BlockSpec((1, H), lambda i: (0, 0)),
          pl.BlockSpec((H, width_out), lambda i: (0, 0)),
          pl.BlockSpec((1, width_out), lambda i: (0, 0)),
      ],
      out_specs=pl.BlockSpec((RB, width_out), lambda i: (i, 0)),
      out_shape=jax.ShapeDtypeStruct((N, width_out), jnp.float32),
  )(s, pprev, dinv, g, be, w, b)


def _pool_body(s_ref, p_ref, dinv_ref, batch_ref, g_ref, be_ref, wm_ref,
               bm_ref, out_ref, acc_ref):
  i = pl.program_id(0)

  @pl.when(i == 0)
  def _():
    acc_ref[...] = jnp.full((G, O), -jnp.inf, jnp.float32)

  s = s_ref[...]
  tot = (s[0] + s[1] + p_ref[...]) * dinv_ref[...]
  z = tot / jnp.sqrt(1.0 + EPS) * g_ref[...] + be_ref[...]
  h = jnp.where(z >= 0, z, 0.01 * z)
  gids = lax.broadcasted_iota(jnp.int32, (PB, G), 1)
  mask = batch_ref[...] == gids
  contrib = jnp.where(mask[:, :, None], h[:, None, :], -jnp.inf)
  acc_ref[...] = jnp.maximum(acc_ref[...], contrib.max(axis=0))

  @pl.when(i == NPB - 1)
  def _():
    pooled = acc_ref[...]
    out_ref[...] = (
        jnp.dot(pooled, wm_ref[...], preferred_element_type=jnp.float32)
        + bm_ref[...])


def _pool(s, p, dinv, batch2d, g, be, wm, bm):
  return pl.pallas_call(
      _pool_body,
      grid=(NPB,),
      in_specs=[
          pl.BlockSpec((NC, PB, O), lambda i: (0, i, 0)),
          pl.BlockSpec((PB, O), lambda i: (i, 0)),
          pl.BlockSpec((PB, 1), lambda i: (i, 0)),
          pl.BlockSpec((PB, 1), lambda i: (i, 0)),
          pl.BlockSpec((1, O), lambda i: (0, 0)),
          pl.BlockSpec((1, O), lambda i: (0, 0)),
          pl.BlockSpec((O, 1), lambda i: (0, 0)),
          pl.BlockSpec((1, 1), lambda i: (0, 0)),
      ],
      out_specs=pl.BlockSpec((G, 1), lambda i: (0, 0)),
      out_shape=jax.ShapeDtypeStruct((G, 1), jnp.float32),
      scratch_shapes=[pltpu.VMEM((G, O), jnp.float32)],
  )(s, p, dinv, batch2d, g, be, wm, bm)


def kernel(x, edge_index, batch, W1, b1, g1, be1, W2, b2, g2, be2,
           W3, b3, g3, be3, Wm, bm):
  pad = EPAD - E
  srcp = jnp.concatenate([edge_index[0], jnp.zeros((pad,), jnp.int32)])
  dstp = jnp.concatenate([edge_index[1], jnp.full((pad,), N, jnp.int32)])

  zeros16 = jnp.zeros((N, 16), jnp.float32)
  zeros_h = jnp.zeros((N, H), jnp.float32)
  zeros_o = jnp.zeros((N, O), jnp.float32)
  ones16 = jnp.ones((CHUNK, 16), jnp.float32)

  degp = _deg_kernel(dstp, ones16, zeros16)
  dinv, p1 = _tc1(degp, x, W1, b1.reshape(1, H))

  s1 = _spmm_h(srcp, dstp, p1, zeros_h)
  p2 = _mid(s1, p1, dinv, g1.reshape(1, H), be1.reshape(1, H),
            W2, b2.reshape(1, H), H)

  s2 = _spmm_h(srcp, dstp, p2, zeros_h)
  p3 = _mid(s2, p2, dinv, g2.reshape(1, H), be2.reshape(1, H),
            W3, b3.reshape(1, O), O)

  s3 = _spmm_o(srcp, dstp, p3, zeros_o)
  out = _pool(s3, p3, dinv, batch.reshape(N, 1),
              g3.reshape(1, O), be3.reshape(1, O), Wm, bm.reshape(1, 1))
  return out


# speedup vs baseline: 8.4077x; 8.4077x over previous
import functools

import jax
import jax.numpy as jnp
from jax import lax
from jax.experimental import pallas as pl
from jax.experimental.pallas import tpu as pltpu
from jax.experimental.pallas import tpu_sc as plsc

N = 10000
E = 320000
D = 128
H = 128
O = 64
G = 64
EPS = 1e-5

NC = 2
NS = 16
CHUNK = 128
NCHUNKS = -(-E // (NC * NS * CHUNK))
EPT = NCHUNKS * CHUNK
EPAD = EPT * NC * NS
NROW = 632
NACC = NROW * NS

@functools.cache
def _mesh():
  return plsc.VectorSubcoreMesh(
      core_axis_name="c", subcore_axis_name="s",
      num_cores=NC, num_subcores=NS)


@functools.cache
def _make_spmm(width):
  @functools.partial(
      pl.kernel,
      out_type=jax.ShapeDtypeStruct((NC, NACC, width), jnp.float32),
      mesh=_mesh(),
      scratch_types=[
          pltpu.VMEM((CHUNK,), jnp.int32),
          pltpu.VMEM((CHUNK,), jnp.int32),
          pltpu.VMEM((CHUNK, width), jnp.float32),
          pltpu.VMEM_SHARED((NACC, width), jnp.float32),
          pltpu.SemaphoreType.DMA,
      ],
  )
  def spmm(src_hbm, dst_hbm, p_hbm, zeros_hbm, out_hbm,
           src_v, dst_v, rows_v, acc, sem):
    cid = lax.axis_index("c")
    sid = lax.axis_index("s")
    pltpu.sync_copy(zeros_hbm.at[pl.ds(sid * NROW, NROW)],
                    acc.at[pl.ds(sid * NROW, NROW)])
    plsc.subcore_barrier()
    base = (cid * NS + sid) * EPT

    def body(i, carry):
      off = base + i * CHUNK
      pltpu.sync_copy(src_hbm.at[pl.ds(off, CHUNK)], src_v)
      pltpu.sync_copy(dst_hbm.at[pl.ds(off, CHUNK)], dst_v)
      pltpu.async_copy(p_hbm.at[src_v], rows_v, sem).wait()
      pltpu.sync_copy(rows_v, acc.at[dst_v], add=True)
      return carry

    lax.fori_loop(0, NCHUNKS, body, 0)
    plsc.subcore_barrier()
    pltpu.sync_copy(acc.at[pl.ds(sid * NROW, NROW)],
                    out_hbm.at[cid, pl.ds(sid * NROW, NROW)])

  return spmm


@functools.cache
def _make_deg():

  @functools.partial(
      pl.kernel,
      out_type=jax.ShapeDtypeStruct((NC, NACC, H), jnp.float32),
      mesh=_mesh(),
      scratch_types=[
          pltpu.VMEM((CHUNK,), jnp.int32),
          pltpu.VMEM((CHUNK, H), jnp.float32),
          pltpu.VMEM_SHARED((NACC, H), jnp.float32),
      ],
  )
  def deg_kernel(dst_hbm, ones_hbm, zeros_hbm, out_hbm, dst_v, ones_v, acc):
    cid = lax.axis_index("c")
    sid = lax.axis_index("s")
    pltpu.sync_copy(ones_hbm, ones_v)
    pltpu.sync_copy(zeros_hbm.at[pl.ds(sid * NROW, NROW)],
                    acc.at[pl.ds(sid * NROW, NROW)])
    plsc.subcore_barrier()
    base = (cid * NS + sid) * EPT

    def body(i, carry):
      off = base + i * CHUNK
      pltpu.sync_copy(dst_hbm.at[pl.ds(off, CHUNK)], dst_v)
      pltpu.sync_copy(ones_v, acc.at[dst_v], add=True)
      return carry

    lax.fori_loop(0, NCHUNKS, body, 0)
    plsc.subcore_barrier()
    pltpu.sync_copy(acc.at[pl.ds(sid * NROW, NROW)],
                    out_hbm.at[cid, pl.ds(sid * NROW, NROW)])

  return deg_kernel


RB = 400
NB = N // RB
PB = 200
NPB = N // PB


def _tc1_body(dp_ref, x_ref, w_ref, b_ref, dinv_ref, p_ref):
  dp = dp_ref[...]
  deg = 1.0 + dp[0, :, 0:1] + dp[1, :, 0:1]
  dinv = lax.rsqrt(deg)
  dinv_ref[...] = dinv
  h = jnp.dot(x_ref[...], w_ref[...], preferred_element_type=jnp.float32)
  p_ref[...] = dinv * (h + b_ref[...])


def _tc1(degp, x, w1, b1):
  return pl.pallas_call(
      _tc1_body,
      grid=(NB,),
      in_specs=[
          pl.BlockSpec((NC, RB, H), lambda i: (0, i, 0)),
          pl.BlockSpec((RB, D), lambda i: (i, 0)),
          pl.BlockSpec((D, H), lambda i: (0, 0)),
          pl.BlockSpec((1, H), lambda i: (0, 0)),
      ],
      out_specs=[
          pl.BlockSpec((RB, 1), lambda i: (i, 0)),
          pl.BlockSpec((RB, H), lambda i: (i, 0)),
      ],
      out_shape=[
          jax.ShapeDtypeStruct((N, 1), jnp.float32),
          jax.ShapeDtypeStruct((N, H), jnp.float32),
      ],
  )(degp, x, w1, b1)


def _mid_body(s_ref, pprev_ref, dinv_ref, g_ref, be_ref, w_ref, b_ref,
              pnext_ref):
  s = s_ref[...]
  dinv = dinv_ref[...]
  tot = (s[0] + s[1] + pprev_ref[...]) * dinv
  z = tot / jnp.sqrt(1.0 + EPS) * g_ref[...] + be_ref[...]
  z = jnp.where(z >= 0, z, 0.01 * z)
  h = jnp.dot(z, w_ref[...], preferred_element_type=jnp.float32)
  pnext_ref[...] = dinv * (h + b_ref[...])


def _mid(s, pprev, dinv, g, be, w, b, width_out):
  return pl.pallas_call(
      _mid_body,
      grid=(NB,),
      in_specs=[
          pl.BlockSpec((NC, RB, H), lambda i: (0, i, 0)),
          pl.BlockSpec((RB, H), lambda i: (i, 0)),
          pl.BlockSpec((RB, 1), lambda i: (i, 0)),
          pl.BlockSpec((1, H), lambda i: (0, 0)),
          pl.BlockSpec((1, H), lambda i: (0, 0)),
          pl.BlockSpec((H, width_out), lambda i: (0, 0)),
          pl.BlockSpec((1, width_out), lambda i: (0, 0)),
      ],
      out_specs=pl.BlockSpec((RB, width_out), lambda i: (i, 0)),
      out_shape=jax.ShapeDtypeStruct((N, width_out), jnp.float32),
  )(s, pprev, dinv, g, be, w, b)


def _pool_body(s_ref, p_ref, dinv_ref, batch_ref, g_ref, be_ref, wm_ref,
               bm_ref, out_ref, acc_ref):
  i = pl.program_id(0)

  @pl.when(i == 0)
  def _():
    acc_ref[...] = jnp.full((G, O), -jnp.inf, jnp.float32)

  s = s_ref[...]
  tot = (s[0, :, :O] + s[1, :, :O] + p_ref[..., :O]) * dinv_ref[...]
  z = tot / jnp.sqrt(1.0 + EPS) * g_ref[...] + be_ref[...]
  h = jnp.where(z >= 0, z, 0.01 * z)
  b = batch_ref[...]
  cols = [
      jnp.max(jnp.where(b == g, h, -jnp.inf), axis=0, keepdims=True)
      for g in range(G)
  ]
  contrib = jnp.concatenate(cols, axis=0)
  acc_ref[...] = jnp.maximum(acc_ref[...], contrib)

  @pl.when(i == NPB - 1)
  def _():
    pooled = acc_ref[...]
    out_ref[...] = (
        jnp.dot(pooled, wm_ref[...], preferred_element_type=jnp.float32)
        + bm_ref[...])


def _pool(s, p, dinv, batch2d, g, be, wm, bm):
  return pl.pallas_call(
      _pool_body,
      grid=(NPB,),
      in_specs=[
          pl.BlockSpec((NC, PB, H), lambda i: (0, i, 0)),
          pl.BlockSpec((PB, H), lambda i: (i, 0)),
          pl.BlockSpec((PB, 1), lambda i: (i, 0)),
          pl.BlockSpec((PB, 1), lambda i: (i, 0)),
          pl.BlockSpec((1, O), lambda i: (0, 0)),
          pl.BlockSpec((1, O), lambda i: (0, 0)),
          pl.BlockSpec((O, 1), lambda i: (0, 0)),
          pl.BlockSpec((1, 1), lambda i: (0, 0)),
      ],
      out_specs=pl.BlockSpec((G, 1), lambda i: (0, 0)),
      out_shape=jax.ShapeDtypeStruct((G, 1), jnp.float32),
      scratch_shapes=[pltpu.VMEM((G, O), jnp.float32)],
  )(s, p, dinv, batch2d, g, be, wm, bm)


def kernel(x, edge_index, batch, W1, b1, g1, be1, W2, b2, g2, be2,
           W3, b3, g3, be3, Wm, bm):
  pad = EPAD - E
  srcp = jnp.concatenate([edge_index[0], jnp.zeros((pad,), jnp.int32)])
  dstp = jnp.concatenate([edge_index[1], jnp.full((pad,), N, jnp.int32)])

  zeros_h = jnp.zeros((NACC, H), jnp.float32)
  ones_c = jnp.ones((CHUNK, H), jnp.float32)
  w3p = jnp.pad(W3, ((0, 0), (0, H - O)))
  b3p = jnp.pad(b3, (0, H - O))

  degp = _make_deg()(dstp, ones_c, zeros_h)
  dinv, p1 = _tc1(degp, x, W1, b1.reshape(1, H))

  s1 = _make_spmm(H)(srcp, dstp, p1, zeros_h)
  p2 = _mid(s1, p1, dinv, g1.reshape(1, H), be1.reshape(1, H),
            W2, b2.reshape(1, H), H)

  s2 = _make_spmm(H)(srcp, dstp, p2, zeros_h)
  p3 = _mid(s2, p2, dinv, g2.reshape(1, H), be2.reshape(1, H),
            w3p, b3p.reshape(1, H), H)

  s3 = _make_spmm(H)(srcp, dstp, p3, zeros_h)
  out = _pool(s3, p3, dinv, batch.reshape(N, 1),
              g3.reshape(1, O), be3.reshape(1, O), Wm, bm.reshape(1, 1))
  return out
